# baseline (speedup 1.0000x reference)
import functools

import jax
import jax.numpy as jnp
from jax import lax
from jax.experimental import pallas as pl
from jax.experimental.pallas import tpu as pltpu
from jax.experimental.pallas import tpu_sc as plsc

CHUNK = 128
NBUF = 8
SUBCORES = 16


def _make_deg_kernel(n_pad, n_chunks):
  chunks_per_tile = n_chunks // (2 * SUBCORES)
  rows_per_tile = n_pad // SUBCORES
  mesh = plsc.VectorSubcoreMesh(core_axis_name="c", subcore_axis_name="s")

  @functools.partial(
      pl.kernel,
      out_type=jax.ShapeDtypeStruct((2 * n_pad,), jnp.float32),
      mesh=mesh,
      scratch_types=[
          pltpu.VMEM_SHARED((n_pad,), jnp.float32),
          pltpu.VMEM((chunks_per_tile, CHUNK), jnp.int32),
          pltpu.VMEM((CHUNK,), jnp.float32),
          pltpu.VMEM((rows_per_tile,), jnp.float32),
          pltpu.SemaphoreType.DMA,
      ],
  )
  def deg_kernel(dst_hbm, out_hbm, acc, dstv, ones_v, zeros_v, ssem):
    c = lax.axis_index("c")
    s = lax.axis_index("s")
    tile = c * SUBCORES + s
    base = s * rows_per_tile

    for i in range(CHUNK // 16):
      ones_v[pl.ds(i * 16, 16)] = jnp.full((16,), 1.0, jnp.float32)
    for i in range(rows_per_tile // 16):
      zeros_v[pl.ds(i * 16, 16)] = jnp.zeros((16,), jnp.float32)

    pltpu.sync_copy(dst_hbm.at[pl.ds(tile * chunks_per_tile, chunks_per_tile)], dstv)
    pltpu.sync_copy(zeros_v, acc.at[pl.ds(base, rows_per_tile)])
    plsc.subcore_barrier()

    @pl.loop(0, chunks_per_tile)
    def _(j):
      pltpu.async_copy(ones_v, acc.at[dstv.at[j]], ssem, add=True)

    @pl.loop(0, chunks_per_tile)
    def _(j):
      pltpu.make_async_copy(ones_v, acc.at[dstv.at[j]], ssem).wait()

    plsc.subcore_barrier()
    pltpu.sync_copy(acc.at[pl.ds(base, rows_per_tile)], zeros_v)
    pltpu.sync_copy(zeros_v, out_hbm.at[pl.ds(c * n_pad + base, rows_per_tile)])

  return deg_kernel


def _make_agg_kernel(n_pad, w2, n_chunks):
  chunks_per_tile = n_chunks // SUBCORES
  rows_per_tile = n_pad // SUBCORES
  mesh = plsc.VectorSubcoreMesh(core_axis_name="c", subcore_axis_name="s")

  @functools.partial(
      pl.kernel,
      out_type=jax.ShapeDtypeStruct((2, n_pad, w2), jnp.float32),
      mesh=mesh,
      scratch_types=[
          pltpu.VMEM_SHARED((n_pad, w2), jnp.float32),
          pltpu.VMEM((chunks_per_tile, CHUNK), jnp.int32),
          pltpu.VMEM((chunks_per_tile, CHUNK), jnp.int32),
          pltpu.VMEM((NBUF, CHUNK, w2), jnp.float32),
          pltpu.SemaphoreType.DMA((NBUF,)),
      ],
      compiler_params=pltpu.CompilerParams(use_tc_tiling_on_sc=False),
  )
  def agg_kernel(hs_l, hs_r, src_hbm, dst_hbm, out_hbm,
                 acc, srcv, dstv, rows, gsem):
    c = lax.axis_index("c")
    s = lax.axis_index("s")
    base = s * rows_per_tile

    pltpu.sync_copy(src_hbm.at[pl.ds(s * chunks_per_tile, chunks_per_tile)], srcv)
    pltpu.sync_copy(dst_hbm.at[pl.ds(s * chunks_per_tile, chunks_per_tile)], dstv)

    def run(hs_hbm):
      pltpu.sync_copy(hs_hbm.at[pl.ds(base, rows_per_tile)],
                      acc.at[pl.ds(base, rows_per_tile)])
      plsc.subcore_barrier()

      for b in range(NBUF):
        pltpu.async_copy(hs_hbm.at[srcv.at[b]], rows.at[b], gsem.at[b])

      @pl.loop(0, chunks_per_tile // NBUF)
      def _(g):
        for b in range(NBUF):
          j = g * NBUF + b
          pltpu.make_async_copy(hs_hbm.at[srcv.at[j]], rows.at[b], gsem.at[b]).wait()
          pltpu.sync_copy(rows.at[b], acc.at[dstv.at[j]], add=True)

          @pl.when(j + NBUF < chunks_per_tile)
          def _():
            pltpu.async_copy(hs_hbm.at[srcv.at[j + NBUF]], rows.at[b], gsem.at[b])

      plsc.subcore_barrier()
      pltpu.sync_copy(acc.at[pl.ds(base, rows_per_tile)],
                      out_hbm.at[c, pl.ds(base, rows_per_tile)])

    @pl.when(c == 0)
    def _():
      run(hs_l)

    @pl.when(c != 0)
    def _():
      run(hs_r)

  return agg_kernel


def _tc_pre_body(dp_ref, x_ref, w1_ref, dinv_ref, hsl_ref, hsr_ref):
  deg = dp_ref[0] + dp_ref[1] + 1.0
  dinv = lax.rsqrt(deg)
  h = jnp.dot(x_ref[...], w1_ref[...], preferred_element_type=jnp.float32)
  hs = h * dinv
  w2 = hsl_ref.shape[-1]
  dinv_ref[...] = dinv
  hsl_ref[...] = hs[:, :w2]
  hsr_ref[...] = hs[:, w2:]


def _tc_mid_body(aggp_ref, dinv_ref, b1_ref, w2_ref, hsl_ref, hsr_ref):
  agg = jnp.concatenate([aggp_ref[0], aggp_ref[1]], axis=-1)
  dinv = dinv_ref[...]
  h1 = jnp.maximum(agg * dinv + b1_ref[...], 0.0)
  hs2 = jnp.dot(h1, w2_ref[...], preferred_element_type=jnp.float32) * dinv
  w2 = hsl_ref.shape[-1]
  hsl_ref[...] = hs2[:, :w2]
  hsr_ref[...] = hs2[:, w2:]


def _tc_post_body(aggp_ref, dinv_ref, b2_ref, z_ref):
  agg = jnp.concatenate([aggp_ref[0], aggp_ref[1]], axis=-1)
  z_ref[...] = agg * dinv_ref[...] + b2_ref[...]


@jax.jit
def kernel(x, edge_index, W1, b1, W2, b2):
  n, in_dim = x.shape
  hidden = W1.shape[1]
  out_dim = W2.shape[1]
  e = edge_index.shape[1]
  h2 = hidden // 2
  o2 = out_dim // 2

  blk = 1024
  n_pad = ((n + blk - 1) // blk + 1) * blk
  grid = n_pad // blk
  per_tile = -(-e // (SUBCORES * CHUNK * 8)) * (CHUNK * 8)
  e_pad = per_tile * SUBCORES
  n_chunks = e_pad // CHUNK

  x_p = jnp.pad(x, ((0, n_pad - n), (0, 0)))
  fill = n + jnp.arange(e_pad - e, dtype=jnp.int32) % (n_pad - n)
  src = jnp.concatenate([edge_index[0], fill]).reshape(n_chunks, CHUNK)
  dst = jnp.concatenate([edge_index[1], fill]).reshape(n_chunks, CHUNK)

  deg_partial = _make_deg_kernel(n_pad, n_chunks)(dst)
  dp = deg_partial.reshape(2, n_pad, 1)

  dinv, hs1l, hs1r = pl.pallas_call(
      _tc_pre_body,
      grid=(grid,),
      in_specs=[
          pl.BlockSpec((2, blk, 1), lambda i: (0, i, 0)),
          pl.BlockSpec((blk, in_dim), lambda i: (i, 0)),
          pl.BlockSpec((in_dim, hidden), lambda i: (0, 0)),
      ],
      out_specs=[
          pl.BlockSpec((blk, 1), lambda i: (i, 0)),
          pl.BlockSpec((blk, h2), lambda i: (i, 0)),
          pl.BlockSpec((blk, h2), lambda i: (i, 0)),
      ],
      out_shape=[
          jax.ShapeDtypeStruct((n_pad, 1), jnp.float32),
          jax.ShapeDtypeStruct((n_pad, h2), jnp.float32),
          jax.ShapeDtypeStruct((n_pad, h2), jnp.float32),
      ],
  )(dp, x_p, W1)

  agg1 = _make_agg_kernel(n_pad, h2, n_chunks)(hs1l, hs1r, src, dst)

  hs2l, hs2r = pl.pallas_call(
      _tc_mid_body,
      grid=(grid,),
      in_specs=[
          pl.BlockSpec((2, blk, h2), lambda i: (0, i, 0)),
          pl.BlockSpec((blk, 1), lambda i: (i, 0)),
          pl.BlockSpec((1, hidden), lambda i: (0, 0)),
          pl.BlockSpec((hidden, out_dim), lambda i: (0, 0)),
      ],
      out_specs=[
          pl.BlockSpec((blk, o2), lambda i: (i, 0)),
          pl.BlockSpec((blk, o2), lambda i: (i, 0)),
      ],
      out_shape=[
          jax.ShapeDtypeStruct((n_pad, o2), jnp.float32),
          jax.ShapeDtypeStruct((n_pad, o2), jnp.float32),
      ],
  )(agg1, dinv, b1.reshape(1, hidden), W2)

  agg2 = _make_agg_kernel(n_pad, o2, n_chunks)(hs2l, hs2r, src, dst)

  blk_o = 1000
  assert n % blk_o == 0
  z = pl.pallas_call(
      _tc_post_body,
      grid=(n // blk_o,),
      in_specs=[
          pl.BlockSpec((2, blk_o, o2), lambda i: (0, i, 0)),
          pl.BlockSpec((blk_o, 1), lambda i: (i, 0)),
          pl.BlockSpec((1, out_dim), lambda i: (0, 0)),
      ],
      out_specs=pl.BlockSpec((blk_o, out_dim), lambda i: (i, 0)),
      out_shape=jax.ShapeDtypeStruct((n, out_dim), jnp.float32),
  )(agg2, dinv, b2.reshape(1, out_dim))

  return z


# speedup vs baseline: 1.0526x; 1.0526x over previous
import functools

import jax
import jax.numpy as jnp
from jax import lax
from jax.experimental import pallas as pl
from jax.experimental.pallas import tpu as pltpu
from jax.experimental.pallas import tpu_sc as plsc

CHUNK = 128
NBUF = 8
SUBCORES = 16


def _make_deg_kernel(n_pad, n_chunks):
  chunks_per_tile = n_chunks // (2 * SUBCORES)
  rows_per_tile = n_pad // SUBCORES
  mesh = plsc.VectorSubcoreMesh(core_axis_name="c", subcore_axis_name="s")

  @functools.partial(
      pl.kernel,
      out_type=jax.ShapeDtypeStruct((2 * n_pad,), jnp.float32),
      mesh=mesh,
      scratch_types=[
          pltpu.VMEM_SHARED((n_pad,), jnp.float32),
          pltpu.VMEM((chunks_per_tile, CHUNK), jnp.int32),
          pltpu.VMEM((CHUNK,), jnp.float32),
          pltpu.VMEM((rows_per_tile,), jnp.float32),
          pltpu.SemaphoreType.DMA,
      ],
  )
  def deg_kernel(dst_hbm, out_hbm, acc, dstv, ones_v, zeros_v, ssem):
    c = lax.axis_index("c")
    s = lax.axis_index("s")
    tile = c * SUBCORES + s
    base = s * rows_per_tile

    for i in range(CHUNK // 16):
      ones_v[pl.ds(i * 16, 16)] = jnp.full((16,), 1.0, jnp.float32)
    for i in range(rows_per_tile // 16):
      zeros_v[pl.ds(i * 16, 16)] = jnp.zeros((16,), jnp.float32)

    pltpu.sync_copy(dst_hbm.at[pl.ds(tile * chunks_per_tile, chunks_per_tile)], dstv)
    pltpu.sync_copy(zeros_v, acc.at[pl.ds(base, rows_per_tile)])
    plsc.subcore_barrier()

    @pl.loop(0, chunks_per_tile)
    def _(j):
      pltpu.async_copy(ones_v, acc.at[dstv.at[j]], ssem, add=True)

    @pl.loop(0, chunks_per_tile)
    def _(j):
      pltpu.make_async_copy(ones_v, acc.at[dstv.at[j]], ssem).wait()

    plsc.subcore_barrier()
    pltpu.sync_copy(acc.at[pl.ds(base, rows_per_tile)], zeros_v)
    pltpu.sync_copy(zeros_v, out_hbm.at[pl.ds(c * n_pad + base, rows_per_tile)])

  return deg_kernel


def _make_agg_kernel(n_pad, w2, n_chunks):
  chunks_per_tile = n_chunks // SUBCORES
  rows_per_tile = n_pad // SUBCORES
  mesh = plsc.VectorSubcoreMesh(core_axis_name="c", subcore_axis_name="s")

  @functools.partial(
      pl.kernel,
      out_type=jax.ShapeDtypeStruct((2, n_pad, w2), jnp.float32),
      mesh=mesh,
      scratch_types=[
          pltpu.VMEM_SHARED((n_pad, w2), jnp.float32),
          pltpu.VMEM((chunks_per_tile, CHUNK), jnp.int32),
          pltpu.VMEM((chunks_per_tile, CHUNK), jnp.int32),
          pltpu.VMEM((NBUF, CHUNK, w2), jnp.float32),
          pltpu.SemaphoreType.DMA((NBUF,)),
      ],
      compiler_params=pltpu.CompilerParams(use_tc_tiling_on_sc=False),
  )
  def agg_kernel(hs_hbm, src_hbm, dst_hbm, out_hbm,
                 acc, srcv, dstv, rows, gsem):
    c = lax.axis_index("c")
    s = lax.axis_index("s")
    base = s * rows_per_tile

    pltpu.sync_copy(src_hbm.at[pl.ds(s * chunks_per_tile, chunks_per_tile)], srcv)
    pltpu.sync_copy(dst_hbm.at[pl.ds(s * chunks_per_tile, chunks_per_tile)], dstv)

    def run(hs_plane):
      pltpu.sync_copy(hs_plane.at[pl.ds(base, rows_per_tile)],
                      acc.at[pl.ds(base, rows_per_tile)])
      plsc.subcore_barrier()

      for b in range(NBUF):
        pltpu.async_copy(hs_plane.at[srcv.at[b]], rows.at[b], gsem.at[b])

      @pl.loop(0, chunks_per_tile // NBUF)
      def _(g):
        for b in range(NBUF):
          j = g * NBUF + b
          pltpu.make_async_copy(hs_plane.at[srcv.at[j]], rows.at[b], gsem.at[b]).wait()
          pltpu.sync_copy(rows.at[b], acc.at[dstv.at[j]], add=True)

          @pl.when(j + NBUF < chunks_per_tile)
          def _():
            pltpu.async_copy(hs_plane.at[srcv.at[j + NBUF]], rows.at[b], gsem.at[b])

      plsc.subcore_barrier()
      pltpu.sync_copy(acc.at[pl.ds(base, rows_per_tile)],
                      out_hbm.at[c, pl.ds(base, rows_per_tile)])

    @pl.when(c == 0)
    def _():
      run(hs_hbm.at[0])

    @pl.when(c != 0)
    def _():
      run(hs_hbm.at[1])

  return agg_kernel


def _tc_pre_body(dp_ref, x_ref, w1_ref, dinv_ref, hs_ref):
  deg = dp_ref[0] + dp_ref[1] + 1.0
  dinv = lax.rsqrt(deg)
  h = jnp.dot(x_ref[...], w1_ref[...], preferred_element_type=jnp.float32)
  hs = h * dinv
  w2 = hs_ref.shape[-1]
  dinv_ref[...] = dinv
  hs_ref[0] = hs[:, :w2]
  hs_ref[1] = hs[:, w2:]


def _tc_mid_body(aggp_ref, dinv_ref, b1_ref, w2_ref, hs_ref):
  agg = jnp.concatenate([aggp_ref[0], aggp_ref[1]], axis=-1)
  dinv = dinv_ref[...]
  h1 = jnp.maximum(agg * dinv + b1_ref[...], 0.0)
  hs2 = jnp.dot(h1, w2_ref[...], preferred_element_type=jnp.float32) * dinv
  w2 = hs_ref.shape[-1]
  hs_ref[0] = hs2[:, :w2]
  hs_ref[1] = hs2[:, w2:]


def _tc_post_body(aggp_ref, dinv_ref, b2_ref, z_ref):
  agg = jnp.concatenate([aggp_ref[0], aggp_ref[1]], axis=-1)
  z_ref[...] = agg * dinv_ref[...] + b2_ref[...]


@jax.jit
def kernel(x, edge_index, W1, b1, W2, b2):
  n, in_dim = x.shape
  hidden = W1.shape[1]
  out_dim = W2.shape[1]
  e = edge_index.shape[1]
  h2 = hidden // 2
  o2 = out_dim // 2

  blk = 1000
  assert n % blk == 0
  grid = n // blk
  n_pad = (n // 128 + 2) * 128
  per_tile = -(-e // (SUBCORES * CHUNK * 8)) * (CHUNK * 8)
  e_pad = per_tile * SUBCORES
  n_chunks = e_pad // CHUNK

  fill = n + jnp.arange(e_pad - e, dtype=jnp.int32) % (n_pad - n)
  src = jnp.concatenate([edge_index[0], fill]).reshape(n_chunks, CHUNK)
  dst = jnp.concatenate([edge_index[1], fill]).reshape(n_chunks, CHUNK)

  deg_partial = _make_deg_kernel(n_pad, n_chunks)(dst)
  dp = deg_partial.reshape(2, n_pad, 1)

  dinv, hs1 = pl.pallas_call(
      _tc_pre_body,
      grid=(grid,),
      in_specs=[
          pl.BlockSpec((2, blk, 1), lambda i: (0, i, 0)),
          pl.BlockSpec((blk, in_dim), lambda i: (i, 0)),
          pl.BlockSpec((in_dim, hidden), lambda i: (0, 0)),
      ],
      out_specs=[
          pl.BlockSpec((blk, 1), lambda i: (i, 0)),
          pl.BlockSpec((2, blk, h2), lambda i: (0, i, 0)),
      ],
      out_shape=[
          jax.ShapeDtypeStruct((n_pad, 1), jnp.float32),
          jax.ShapeDtypeStruct((2, n_pad, h2), jnp.float32),
      ],
  )(dp, x, W1)

  agg1 = _make_agg_kernel(n_pad, h2, n_chunks)(hs1, src, dst)

  hs2 = pl.pallas_call(
      _tc_mid_body,
      grid=(grid,),
      in_specs=[
          pl.BlockSpec((2, blk, h2), lambda i: (0, i, 0)),
          pl.BlockSpec((blk, 1), lambda i: (i, 0)),
          pl.BlockSpec((1, hidden), lambda i: (0, 0)),
          pl.BlockSpec((hidden, out_dim), lambda i: (0, 0)),
      ],
      out_specs=pl.BlockSpec((2, blk, o2), lambda i: (0, i, 0)),
      out_shape=jax.ShapeDtypeStruct((2, n_pad, o2), jnp.float32),
  )(agg1, dinv, b1.reshape(1, hidden), W2)

  agg2 = _make_agg_kernel(n_pad, o2, n_chunks)(hs2, src, dst)

  z = pl.pallas_call(
      _tc_post_body,
      grid=(grid,),
      in_specs=[
          pl.BlockSpec((2, blk, o2), lambda i: (0, i, 0)),
          pl.BlockSpec((blk, 1), lambda i: (i, 0)),
          pl.BlockSpec((1, out_dim), lambda i: (0, 0)),
      ],
      out_specs=pl.BlockSpec((blk, out_dim), lambda i: (i, 0)),
      out_shape=jax.ShapeDtypeStruct((n, out_dim), jnp.float32),
  )(agg2, dinv, b2.reshape(1, out_dim))

  return z


# speedup vs baseline: 1.0528x; 1.0002x over previous
import functools

import jax
import jax.numpy as jnp
from jax import lax
from jax.experimental import pallas as pl
from jax.experimental.pallas import tpu as pltpu
from jax.experimental.pallas import tpu_sc as plsc

CHUNK = 128
NBUF = 8
SUBCORES = 16


def _make_deg_kernel(n_pad, n_chunks):
  chunks_per_tile = n_chunks // (2 * SUBCORES)
  rows_per_tile = n_pad // SUBCORES
  mesh = plsc.VectorSubcoreMesh(core_axis_name="c", subcore_axis_name="s")

  @functools.partial(
      pl.kernel,
      out_type=jax.ShapeDtypeStruct((2 * n_pad,), jnp.float32),
      mesh=mesh,
      scratch_types=[
          pltpu.VMEM_SHARED((n_pad,), jnp.float32),
          pltpu.VMEM((chunks_per_tile, CHUNK), jnp.int32),
          pltpu.VMEM((CHUNK,), jnp.float32),
          pltpu.VMEM((rows_per_tile,), jnp.float32),
          pltpu.SemaphoreType.DMA,
      ],
      compiler_params=pltpu.CompilerParams(use_tc_tiling_on_sc=False),
  )
  def deg_kernel(dst_hbm, out_hbm, acc, dstv, ones_v, zeros_v, ssem):
    c = lax.axis_index("c")
    s = lax.axis_index("s")
    tile = c * SUBCORES + s
    base = s * rows_per_tile

    for i in range(CHUNK // 16):
      ones_v[pl.ds(i * 16, 16)] = jnp.full((16,), 1.0, jnp.float32)
    for i in range(rows_per_tile // 16):
      zeros_v[pl.ds(i * 16, 16)] = jnp.zeros((16,), jnp.float32)

    pltpu.sync_copy(dst_hbm.at[pl.ds(tile * chunks_per_tile, chunks_per_tile)], dstv)
    pltpu.sync_copy(zeros_v, acc.at[pl.ds(base, rows_per_tile)])
    plsc.subcore_barrier()

    @pl.loop(0, chunks_per_tile)
    def _(j):
      pltpu.async_copy(ones_v, acc.at[dstv.at[j]], ssem, add=True)

    @pl.loop(0, chunks_per_tile)
    def _(j):
      pltpu.make_async_copy(ones_v, acc.at[dstv.at[j]], ssem).wait()

    plsc.subcore_barrier()
    pltpu.sync_copy(acc.at[pl.ds(base, rows_per_tile)], zeros_v)
    pltpu.sync_copy(zeros_v, out_hbm.at[pl.ds(c * n_pad + base, rows_per_tile)])

  return deg_kernel


def _make_agg_kernel(n_pad, w2, n_chunks):
  chunks_per_tile = n_chunks // SUBCORES
  rows_per_tile = n_pad // SUBCORES
  mesh = plsc.VectorSubcoreMesh(core_axis_name="c", subcore_axis_name="s")

  @functools.partial(
      pl.kernel,
      out_type=jax.ShapeDtypeStruct((2, n_pad, w2), jnp.float32),
      mesh=mesh,
      scratch_types=[
          pltpu.VMEM_SHARED((n_pad, w2), jnp.float32),
          pltpu.VMEM((chunks_per_tile, CHUNK), jnp.int32),
          pltpu.VMEM((chunks_per_tile, CHUNK), jnp.int32),
          pltpu.VMEM((NBUF, CHUNK, w2), jnp.float32),
          pltpu.SemaphoreType.DMA((NBUF,)),
      ],
      compiler_params=pltpu.CompilerParams(use_tc_tiling_on_sc=False),
  )
  def agg_kernel(hs_hbm, src_hbm, dst_hbm, out_hbm,
                 acc, srcv, dstv, rows, gsem):
    c = lax.axis_index("c")
    s = lax.axis_index("s")
    base = s * rows_per_tile

    pltpu.sync_copy(src_hbm.at[pl.ds(s * chunks_per_tile, chunks_per_tile)], srcv)
    pltpu.sync_copy(dst_hbm.at[pl.ds(s * chunks_per_tile, chunks_per_tile)], dstv)

    def run(hs_plane):
      pltpu.sync_copy(hs_plane.at[pl.ds(base, rows_per_tile)],
                      acc.at[pl.ds(base, rows_per_tile)])
      plsc.subcore_barrier()

      for b in range(NBUF):
        pltpu.async_copy(hs_plane.at[srcv.at[b]], rows.at[b], gsem.at[b])

      @pl.loop(0, chunks_per_tile // NBUF)
      def _(g):
        for b in range(NBUF):
          j = g * NBUF + b
          pltpu.make_async_copy(hs_plane.at[srcv.at[j]], rows.at[b], gsem.at[b]).wait()
          pltpu.sync_copy(rows.at[b], acc.at[dstv.at[j]], add=True)

          @pl.when(j + NBUF < chunks_per_tile)
          def _():
            pltpu.async_copy(hs_plane.at[srcv.at[j + NBUF]], rows.at[b], gsem.at[b])

      plsc.subcore_barrier()
      pltpu.sync_copy(acc.at[pl.ds(base, rows_per_tile)],
                      out_hbm.at[c, pl.ds(base, rows_per_tile)])

    @pl.when(c == 0)
    def _():
      run(hs_hbm.at[0])

    @pl.when(c != 0)
    def _():
      run(hs_hbm.at[1])

  return agg_kernel


def _tc_pre_body(dp_ref, x_ref, w1_ref, dinv_ref, hs_ref):
  deg = dp_ref[0] + dp_ref[1] + 1.0
  dinv = lax.rsqrt(deg)
  h = jnp.dot(x_ref[...], w1_ref[...], preferred_element_type=jnp.float32)
  hs = h * dinv
  w2 = hs_ref.shape[-1]
  dinv_ref[...] = dinv
  hs_ref[0] = hs[:, :w2]
  hs_ref[1] = hs[:, w2:]


def _tc_mid_body(aggp_ref, dinv_ref, b1_ref, w2_ref, hs_ref):
  agg = jnp.concatenate([aggp_ref[0], aggp_ref[1]], axis=-1)
  dinv = dinv_ref[...]
  h1 = jnp.maximum(agg * dinv + b1_ref[...], 0.0)
  hs2 = jnp.dot(h1, w2_ref[...], preferred_element_type=jnp.float32) * dinv
  w2 = hs_ref.shape[-1]
  hs_ref[0] = hs2[:, :w2]
  hs_ref[1] = hs2[:, w2:]


def _tc_post_body(aggp_ref, dinv_ref, b2_ref, z_ref):
  agg = jnp.concatenate([aggp_ref[0], aggp_ref[1]], axis=-1)
  z_ref[...] = agg * dinv_ref[...] + b2_ref[...]


@jax.jit
def kernel(x, edge_index, W1, b1, W2, b2):
  n, in_dim = x.shape
  hidden = W1.shape[1]
  out_dim = W2.shape[1]
  e = edge_index.shape[1]
  h2 = hidden // 2
  o2 = out_dim // 2

  blk = 1000
  assert n % blk == 0
  grid = n // blk
  n_pad = (n // 128 + 2) * 128
  per_tile = -(-e // (SUBCORES * CHUNK * 8)) * (CHUNK * 8)
  e_pad = per_tile * SUBCORES
  n_chunks = e_pad // CHUNK

  fill = n + jnp.arange(e_pad - e, dtype=jnp.int32) % (n_pad - n)
  src = jnp.concatenate([edge_index[0], fill]).reshape(n_chunks, CHUNK)
  dst = jnp.concatenate([edge_index[1], fill]).reshape(n_chunks, CHUNK)

  deg_partial = _make_deg_kernel(n_pad, n_chunks)(dst)
  dp = deg_partial.reshape(2, n_pad, 1)

  dinv, hs1 = pl.pallas_call(
      _tc_pre_body,
      grid=(grid,),
      in_specs=[
          pl.BlockSpec((2, blk, 1), lambda i: (0, i, 0)),
          pl.BlockSpec((blk, in_dim), lambda i: (i, 0)),
          pl.BlockSpec((in_dim, hidden), lambda i: (0, 0)),
      ],
      out_specs=[
          pl.BlockSpec((blk, 1), lambda i: (i, 0)),
          pl.BlockSpec((2, blk, h2), lambda i: (0, i, 0)),
      ],
      out_shape=[
          jax.ShapeDtypeStruct((n_pad, 1), jnp.float32),
          jax.ShapeDtypeStruct((2, n_pad, h2), jnp.float32),
      ],
  )(dp, x, W1)

  agg1 = _make_agg_kernel(n_pad, h2, n_chunks)(hs1, src, dst)

  hs2 = pl.pallas_call(
      _tc_mid_body,
      grid=(grid,),
      in_specs=[
          pl.BlockSpec((2, blk, h2), lambda i: (0, i, 0)),
          pl.BlockSpec((blk, 1), lambda i: (i, 0)),
          pl.BlockSpec((1, hidden), lambda i: (0, 0)),
          pl.BlockSpec((hidden, out_dim), lambda i: (0, 0)),
      ],
      out_specs=pl.BlockSpec((2, blk, o2), lambda i: (0, i, 0)),
      out_shape=jax.ShapeDtypeStruct((2, n_pad, o2), jnp.float32),
  )(agg1, dinv, b1.reshape(1, hidden), W2)

  agg2 = _make_agg_kernel(n_pad, o2, n_chunks)(hs2, src, dst)

  z = pl.pallas_call(
      _tc_post_body,
      grid=(grid,),
      in_specs=[
          pl.BlockSpec((2, blk, o2), lambda i: (0, i, 0)),
          pl.BlockSpec((blk, 1), lambda i: (i, 0)),
          pl.BlockSpec((1, out_dim), lambda i: (0, 0)),
      ],
      out_specs=pl.BlockSpec((blk, out_dim), lambda i: (i, 0)),
      out_shape=jax.ShapeDtypeStruct((n, out_dim), jnp.float32),
  )(agg2, dinv, b2.reshape(1, out_dim))

  return z


# speedup vs baseline: 1.1169x; 1.0609x over previous
import functools

import jax
import jax.numpy as jnp
from jax import lax
from jax.experimental import pallas as pl
from jax.experimental.pallas import tpu as pltpu
from jax.experimental.pallas import tpu_sc as plsc

CHUNK = 128
NBUF = 8
SUBCORES = 16


def _make_deg_kernel(n_pad, n_chunks):
  chunks_per_tile = n_chunks // (2 * SUBCORES)
  rows_per_tile = n_pad // SUBCORES
  mesh = plsc.VectorSubcoreMesh(core_axis_name="c", subcore_axis_name="s")

  @functools.partial(
      pl.kernel,
      out_type=(jax.ShapeDtypeStruct((n_pad,), jnp.float32),
                jax.ShapeDtypeStruct((n_pad,), jnp.float32)),
      mesh=mesh,
      scratch_types=[
          pltpu.VMEM_SHARED((n_pad,), jnp.float32),
          pltpu.VMEM((chunks_per_tile, CHUNK), jnp.int32),
          pltpu.VMEM((CHUNK,), jnp.float32),
          pltpu.VMEM((rows_per_tile,), jnp.float32),
          pltpu.SemaphoreType.DMA,
      ],
      compiler_params=pltpu.CompilerParams(use_tc_tiling_on_sc=False),
  )
  def deg_kernel(dst_hbm, out0_hbm, out1_hbm, acc, dstv, ones_v, zeros_v, ssem):
    c = lax.axis_index("c")
    s = lax.axis_index("s")
    tile = c * SUBCORES + s
    base = s * rows_per_tile

    for i in range(CHUNK // 16):
      ones_v[pl.ds(i * 16, 16)] = jnp.full((16,), 1.0, jnp.float32)
    for i in range(rows_per_tile // 16):
      zeros_v[pl.ds(i * 16, 16)] = jnp.zeros((16,), jnp.float32)

    pltpu.sync_copy(dst_hbm.at[pl.ds(tile * chunks_per_tile, chunks_per_tile)], dstv)
    pltpu.sync_copy(zeros_v, acc.at[pl.ds(base, rows_per_tile)])
    plsc.subcore_barrier()

    @pl.loop(0, chunks_per_tile)
    def _(j):
      pltpu.async_copy(ones_v, acc.at[dstv.at[j]], ssem, add=True)

    @pl.loop(0, chunks_per_tile)
    def _(j):
      pltpu.make_async_copy(ones_v, acc.at[dstv.at[j]], ssem).wait()

    plsc.subcore_barrier()
    pltpu.sync_copy(acc.at[pl.ds(base, rows_per_tile)], zeros_v)

    @pl.when(c == 0)
    def _():
      pltpu.sync_copy(zeros_v, out0_hbm.at[pl.ds(base, rows_per_tile)])

    @pl.when(c != 0)
    def _():
      pltpu.sync_copy(zeros_v, out1_hbm.at[pl.ds(base, rows_per_tile)])

  return deg_kernel


def _make_agg_kernel(n_pad, w2, n_chunks):
  chunks_per_tile = n_chunks // SUBCORES
  rows_per_tile = n_pad // SUBCORES
  mesh = plsc.VectorSubcoreMesh(core_axis_name="c", subcore_axis_name="s")

  @functools.partial(
      pl.kernel,
      out_type=(jax.ShapeDtypeStruct((n_pad, w2), jnp.float32),
                jax.ShapeDtypeStruct((n_pad, w2), jnp.float32)),
      mesh=mesh,
      scratch_types=[
          pltpu.VMEM_SHARED((n_pad, w2), jnp.float32),
          pltpu.VMEM((chunks_per_tile, CHUNK), jnp.int32),
          pltpu.VMEM((chunks_per_tile, CHUNK), jnp.int32),
          pltpu.VMEM((NBUF, CHUNK, w2), jnp.float32),
          pltpu.SemaphoreType.DMA((NBUF,)),
      ],
      compiler_params=pltpu.CompilerParams(use_tc_tiling_on_sc=False),
  )
  def agg_kernel(hs_l, hs_r, src_hbm, dst_hbm, out0_hbm, out1_hbm,
                 acc, srcv, dstv, rows, gsem):
    c = lax.axis_index("c")
    s = lax.axis_index("s")
    base = s * rows_per_tile

    pltpu.sync_copy(src_hbm.at[pl.ds(s * chunks_per_tile, chunks_per_tile)], srcv)
    pltpu.sync_copy(dst_hbm.at[pl.ds(s * chunks_per_tile, chunks_per_tile)], dstv)

    def run(hs_plane, out_hbm):
      pltpu.sync_copy(hs_plane.at[pl.ds(base, rows_per_tile)],
                      acc.at[pl.ds(base, rows_per_tile)])
      plsc.subcore_barrier()

      for b in range(NBUF):
        pltpu.async_copy(hs_plane.at[srcv.at[b]], rows.at[b], gsem.at[b])

      @pl.loop(0, chunks_per_tile // NBUF)
      def _(g):
        for b in range(NBUF):
          j = g * NBUF + b
          pltpu.make_async_copy(hs_plane.at[srcv.at[j]], rows.at[b], gsem.at[b]).wait()
          pltpu.sync_copy(rows.at[b], acc.at[dstv.at[j]], add=True)

          @pl.when(j + NBUF < chunks_per_tile)
          def _():
            pltpu.async_copy(hs_plane.at[srcv.at[j + NBUF]], rows.at[b], gsem.at[b])

      plsc.subcore_barrier()
      pltpu.sync_copy(acc.at[pl.ds(base, rows_per_tile)],
                      out_hbm.at[pl.ds(base, rows_per_tile)])

    @pl.when(c == 0)
    def _():
      run(hs_l, out0_hbm)

    @pl.when(c != 0)
    def _():
      run(hs_r, out1_hbm)

  return agg_kernel


def _fold128(v):
  blk, w = v.shape
  r = 128 // w
  v3 = v.reshape(blk // r, r, w)
  return jnp.concatenate([v3[:, j, :] for j in range(r)], axis=-1)


def _unfold128(m, w):
  rows = m.shape[0]
  r = 128 // w
  parts = [m[:, j * w:(j + 1) * w] for j in range(r)]
  return jnp.stack(parts, axis=1).reshape(rows * r, w)


def _tc_pre_body(dp0_ref, dp1_ref, x_ref, w1_ref, dinv_ref, hsl_ref, hsr_ref):
  blk = x_ref.shape[0]
  deg = dp0_ref[...] + dp1_ref[...] + 1.0
  dinv = lax.rsqrt(deg).reshape(blk, 1)
  h = jnp.dot(x_ref[...], w1_ref[...], preferred_element_type=jnp.float32)
  hs = h * dinv
  w2 = h.shape[-1] // 2
  dinv_ref[...] = dinv.reshape(blk)
  hsl_ref[...] = _fold128(hs[:, :w2])
  hsr_ref[...] = _fold128(hs[:, w2:])


def _tc_mid_body(al_ref, ar_ref, dinv_ref, b1_ref, w2_ref, hsl_ref, hsr_ref):
  blk = dinv_ref.shape[0]
  hidden = b1_ref.shape[-1]
  h2 = hidden // 2
  agg = jnp.concatenate([_unfold128(al_ref[...], h2),
                         _unfold128(ar_ref[...], h2)], axis=-1)
  dinv = dinv_ref[...].reshape(blk, 1)
  h1 = jnp.maximum(agg * dinv + b1_ref[...], 0.0)
  hs2 = jnp.dot(h1, w2_ref[...], preferred_element_type=jnp.float32) * dinv
  o2 = hs2.shape[-1] // 2
  hsl_ref[...] = _fold128(hs2[:, :o2])
  hsr_ref[...] = _fold128(hs2[:, o2:])


def _tc_post_body(al_ref, ar_ref, dinv_ref, b2_ref, z_ref):
  blk = dinv_ref.shape[0]
  o2 = b2_ref.shape[-1] // 2
  agg = jnp.concatenate([_unfold128(al_ref[...], o2),
                         _unfold128(ar_ref[...], o2)], axis=-1)
  z_ref[...] = agg * dinv_ref[...].reshape(blk, 1) + b2_ref[...]


@jax.jit
def kernel(x, edge_index, W1, b1, W2, b2):
  n, in_dim = x.shape
  hidden = W1.shape[1]
  out_dim = W2.shape[1]
  e = edge_index.shape[1]
  h2 = hidden // 2
  o2 = out_dim // 2

  blk = 1024
  n_pad = ((n + blk - 1) // blk + 1) * blk
  grid = n_pad // blk
  per_tile = -(-e // (SUBCORES * CHUNK * 8)) * (CHUNK * 8)
  e_pad = per_tile * SUBCORES
  n_chunks = e_pad // CHUNK

  x_p = jnp.pad(x, ((0, n_pad - n), (0, 0)))
  fill = n + jnp.arange(e_pad - e, dtype=jnp.int32) % (n_pad - n)
  src = jnp.concatenate([edge_index[0], fill]).reshape(n_chunks, CHUNK)
  dst = jnp.concatenate([edge_index[1], fill]).reshape(n_chunks, CHUNK)

  dp0, dp1 = _make_deg_kernel(n_pad, n_chunks)(dst)

  dinv, hs1l, hs1r = pl.pallas_call(
      _tc_pre_body,
      grid=(grid,),
      in_specs=[
          pl.BlockSpec((blk,), lambda i: (i,)),
          pl.BlockSpec((blk,), lambda i: (i,)),
          pl.BlockSpec((blk, in_dim), lambda i: (i, 0)),
          pl.BlockSpec((in_dim, hidden), lambda i: (0, 0)),
      ],
      out_specs=[
          pl.BlockSpec((blk,), lambda i: (i,)),
          pl.BlockSpec((blk * h2 // 128, 128), lambda i: (i, 0)),
          pl.BlockSpec((blk * h2 // 128, 128), lambda i: (i, 0)),
      ],
      out_shape=[
          jax.ShapeDtypeStruct((n_pad,), jnp.float32),
          jax.ShapeDtypeStruct((n_pad * h2 // 128, 128), jnp.float32),
          jax.ShapeDtypeStruct((n_pad * h2 // 128, 128), jnp.float32),
      ],
  )(dp0, dp1, x_p, W1)

  a1l, a1r = _make_agg_kernel(n_pad, h2, n_chunks)(
      hs1l.reshape(n_pad, h2), hs1r.reshape(n_pad, h2), src, dst)

  hs2l, hs2r = pl.pallas_call(
      _tc_mid_body,
      grid=(grid,),
      in_specs=[
          pl.BlockSpec((blk * h2 // 128, 128), lambda i: (i, 0)),
          pl.BlockSpec((blk * h2 // 128, 128), lambda i: (i, 0)),
          pl.BlockSpec((blk,), lambda i: (i,)),
          pl.BlockSpec((1, hidden), lambda i: (0, 0)),
          pl.BlockSpec((hidden, out_dim), lambda i: (0, 0)),
      ],
      out_specs=[
          pl.BlockSpec((blk * o2 // 128, 128), lambda i: (i, 0)),
          pl.BlockSpec((blk * o2 // 128, 128), lambda i: (i, 0)),
      ],
      out_shape=[
          jax.ShapeDtypeStruct((n_pad * o2 // 128, 128), jnp.float32),
          jax.ShapeDtypeStruct((n_pad * o2 // 128, 128), jnp.float32),
      ],
  )(a1l.reshape(n_pad * h2 // 128, 128), a1r.reshape(n_pad * h2 // 128, 128),
    dinv, b1.reshape(1, hidden), W2)

  a2l, a2r = _make_agg_kernel(n_pad, o2, n_chunks)(
      hs2l.reshape(n_pad, o2), hs2r.reshape(n_pad, o2), src, dst)

  z_p = pl.pallas_call(
      _tc_post_body,
      grid=(grid,),
      in_specs=[
          pl.BlockSpec((blk * o2 // 128, 128), lambda i: (i, 0)),
          pl.BlockSpec((blk * o2 // 128, 128), lambda i: (i, 0)),
          pl.BlockSpec((blk,), lambda i: (i,)),
          pl.BlockSpec((1, out_dim), lambda i: (0, 0)),
      ],
      out_specs=pl.BlockSpec((blk, out_dim), lambda i: (i, 0)),
      out_shape=jax.ShapeDtypeStruct((n_pad, out_dim), jnp.float32),
  )(a2l.reshape(n_pad * o2 // 128, 128), a2r.reshape(n_pad * o2 // 128, 128),
    dinv, b2.reshape(1, out_dim))

  return z_p[:n]
